# Initial kernel scaffold; baseline (speedup 1.0000x reference)
import jax
import jax.numpy as jnp
from jax import lax
from jax.experimental import pallas as pl
from jax.experimental.pallas import tpu as pltpu
from jax.experimental.pallas import tpu_sc as plsc

N = 10000
E = 160000
IN_DIM = 1433
HID = 512
OUT_DIM = 7

NTILE = 16
NH = 10240
ROWS_PT = NH // NTILE
EPAD = 163840
BE = 128
NB_C = EPAD // NTILE // BE
NB_W = EPAD // (2 * NTILE) // BE
CW = 128
NCH = HID // CW
CW2 = 16

_mesh = plsc.VectorSubcoreMesh(core_axis_name="c", subcore_axis_name="s")

f32 = jnp.float32
i32 = jnp.int32


def _zfill(ref, n):
    def body(j, _):
        ref[pl.ds(j * 16, 16)] = jnp.zeros((16,), f32)
        return 0
    lax.fori_loop(0, n // 16, body, 0)


def _deg_body(col_hbm, deg_hbm, colv, ones, zbuf, hist):
    c = lax.axis_index("c")
    s = lax.axis_index("s")
    w = c * NTILE + s
    def fo(j, _):
        ones[pl.ds(j * 16, 16)] = jnp.full((16,), 1.0, f32)
        return 0
    lax.fori_loop(0, BE // 16, fo, 0)
    _zfill(zbuf, ROWS_PT)
    pltpu.sync_copy(zbuf, hist.at[pl.ds(s * ROWS_PT, ROWS_PT)])
    pltpu.sync_copy(col_hbm.at[w], colv)
    plsc.subcore_barrier()
    def body(b, _):
        pltpu.sync_copy(ones, hist.at[colv.at[b]], add=True)
        return 0
    lax.fori_loop(0, NB_W, body, 0)
    plsc.subcore_barrier()
    pltpu.sync_copy(hist.at[pl.ds(s * ROWS_PT, ROWS_PT)],
                    deg_hbm.at[pl.ds(c * NH + s * ROWS_PT, ROWS_PT)])


def _deg_call(colA):
    return pl.kernel(
        _deg_body,
        out_type=jax.ShapeDtypeStruct((2 * NH,), f32),
        mesh=_mesh,
        scratch_types=[
            pltpu.VMEM((NB_W, BE), i32),
            pltpu.VMEM((BE,), f32),
            pltpu.VMEM((ROWS_PT,), f32),
            pltpu.VMEM_SHARED((NH,), f32),
        ],
    )(colA)


def _mm1_body(x_ref, w_ref, deg_ref, o_ref):
    d = deg_ref[0, :] + deg_ref[1, :] + 1.0
    dinv = lax.rsqrt(d)
    acc = jnp.dot(x_ref[...], w_ref[...], preferred_element_type=f32)
    o_ref[0] = acc * dinv[:, None]


def _mm1_call(x, W1, deg2):
    bn = 2000
    return pl.pallas_call(
        _mm1_body,
        grid=(N // bn, NCH),
        in_specs=[
            pl.BlockSpec((bn, IN_DIM), lambda i, k: (i, 0)),
            pl.BlockSpec((IN_DIM, CW), lambda i, k: (0, k)),
            pl.BlockSpec((2, bn), lambda i, k: (0, i)),
        ],
        out_specs=pl.BlockSpec((1, bn, CW), lambda i, k: (k, i, 0)),
        out_shape=jax.ShapeDtypeStruct((NCH, NH, CW), f32),
    )(x, W1, deg2)


def _prop1_body(g1_hbm, row_hbm, col_hbm, s1_hbm,
                rowv, colv, idxg, gbuf, zbuf, acc, sem):
    c = lax.axis_index("c")
    s = lax.axis_index("s")
    pltpu.sync_copy(row_hbm.at[s], rowv)
    pltpu.sync_copy(col_hbm.at[s], colv)
    def fz(r, _):
        def fz2(j, _):
            zbuf[r, pl.ds(j * 16, 16)] = jnp.zeros((16,), f32)
            return 0
        lax.fori_loop(0, CW // 16, fz2, 0)
        return 0
    lax.fori_loop(0, 64, fz, 0)
    for kl in range(2):
        k = c * 2 + kl
        off = k * NH
        def zc(q, _):
            pltpu.sync_copy(zbuf, acc.at[pl.ds(s * ROWS_PT + q * 64, 64)])
            return 0
        lax.fori_loop(0, ROWS_PT // 64, zc, 0)
        plsc.subcore_barrier()
        def body(b, _):
            for j in range(BE // 16):
                idxg[pl.ds(j * 16, 16)] = rowv[b, pl.ds(j * 16, 16)] + off
            pltpu.async_copy(g1_hbm.at[idxg], gbuf, sem).wait()
            pltpu.sync_copy(gbuf, acc.at[colv.at[b]], add=True)
            return 0
        lax.fori_loop(0, NB_C, body, 0)
        plsc.subcore_barrier()
        pltpu.sync_copy(acc.at[pl.ds(s * ROWS_PT, ROWS_PT)],
                        s1_hbm.at[pl.ds(off + s * ROWS_PT, ROWS_PT)])


def _prop1_call(g1f, rowC, colC):
    return pl.kernel(
        _prop1_body,
        out_type=jax.ShapeDtypeStruct((NCH * NH, CW), f32),
        mesh=_mesh,
        scratch_types=[
            pltpu.VMEM((NB_C, BE), i32),
            pltpu.VMEM((NB_C, BE), i32),
            pltpu.VMEM((BE,), i32),
            pltpu.VMEM((BE, CW), f32),
            pltpu.VMEM((64, CW), f32),
            pltpu.VMEM_SHARED((NH, CW), f32),
            pltpu.SemaphoreType.DMA,
        ],
    )(g1f, rowC, colC)


def _mm2_body(s1_ref, g1_ref, deg_ref, b1_ref, w2_ref, o_ref):
    d = deg_ref[0, :] + deg_ref[1, :] + 1.0
    dinv = lax.rsqrt(d)
    bn = d.shape[0]
    acc = jnp.zeros((bn, CW2), f32)
    for k in range(NCH):
        t = dinv[:, None] * (s1_ref[k] + g1_ref[k]) + b1_ref[k][None, :]
        t = jnp.maximum(t, 0.0)
        acc = acc + jnp.dot(t, w2_ref[k], preferred_element_type=f32)
    o_ref[...] = acc * dinv[:, None]


def _mm2_call(s1, g1, deg2, b1r, W2p):
    bn = 400
    return pl.pallas_call(
        _mm2_body,
        grid=(N // bn,),
        in_specs=[
            pl.BlockSpec((NCH, bn, CW), lambda i: (0, i, 0)),
            pl.BlockSpec((NCH, bn, CW), lambda i: (0, i, 0)),
            pl.BlockSpec((2, bn), lambda i: (0, i)),
            pl.BlockSpec((NCH, CW), lambda i: (0, 0)),
            pl.BlockSpec((NCH, CW, CW2), lambda i: (0, 0, 0)),
        ],
        out_specs=pl.BlockSpec((bn, CW2), lambda i: (i, 0)),
        out_shape=jax.ShapeDtypeStruct((N, CW2), f32),
    )(s1, g1, deg2, b1r, W2p)


def _prop2_body(g2_hbm, row_hbm, col_hbm, s2_hbm,
                rowv, colv, gbuf, zbuf, acc, sem):
    c = lax.axis_index("c")
    s = lax.axis_index("s")
    w = c * NTILE + s
    pltpu.sync_copy(row_hbm.at[w], rowv)
    pltpu.sync_copy(col_hbm.at[w], colv)
    def fz(r, _):
        zbuf[r, pl.ds(0, 16)] = jnp.zeros((16,), f32)
        return 0
    lax.fori_loop(0, 64, fz, 0)
    def zc(q, _):
        pltpu.sync_copy(zbuf, acc.at[pl.ds(s * ROWS_PT + q * 64, 64)])
        return 0
    lax.fori_loop(0, ROWS_PT // 64, zc, 0)
    plsc.subcore_barrier()
    def body(b, _):
        pltpu.async_copy(g2_hbm.at[rowv.at[b]], gbuf, sem).wait()
        pltpu.sync_copy(gbuf, acc.at[colv.at[b]], add=True)
        return 0
    lax.fori_loop(0, NB_W, body, 0)
    plsc.subcore_barrier()
    pltpu.sync_copy(acc.at[pl.ds(s * ROWS_PT, ROWS_PT)],
                    s2_hbm.at[pl.ds(c * NH + s * ROWS_PT, ROWS_PT)])


def _prop2_call(g2, rowE, colE):
    return pl.kernel(
        _prop2_body,
        out_type=jax.ShapeDtypeStruct((2 * NH, CW2), f32),
        mesh=_mesh,
        scratch_types=[
            pltpu.VMEM((NB_W, BE), i32),
            pltpu.VMEM((NB_W, BE), i32),
            pltpu.VMEM((BE, CW2), f32),
            pltpu.VMEM((64, CW2), f32),
            pltpu.VMEM_SHARED((NH, CW2), f32),
            pltpu.SemaphoreType.DMA,
        ],
    )(g2, rowE, colE)


def _fin_body(s2_ref, g2_ref, deg_ref, b2_ref, o_ref):
    d = deg_ref[0, :] + deg_ref[1, :] + 1.0
    dinv = lax.rsqrt(d)
    o_ref[...] = dinv[:, None] * (s2_ref[0] + s2_ref[1] + g2_ref[...]) \
        + b2_ref[0][None, :]


def _fin_call(s2, g2, deg2, b2p):
    bn = 400
    return pl.pallas_call(
        _fin_body,
        grid=(N // bn,),
        in_specs=[
            pl.BlockSpec((2, bn, CW2), lambda i: (0, i, 0)),
            pl.BlockSpec((bn, CW2), lambda i: (i, 0)),
            pl.BlockSpec((2, bn), lambda i: (0, i)),
            pl.BlockSpec((1, CW2), lambda i: (0, 0)),
        ],
        out_specs=pl.BlockSpec((bn, CW2), lambda i: (i, 0)),
        out_shape=jax.ShapeDtypeStruct((N, CW2), f32),
    )(s2, g2, deg2, b2p)


def kernel(x, edge_index, W1, b1, W2, b2):
    npad = EPAD - E
    rowp = jnp.concatenate([edge_index[0], jnp.zeros((npad,), i32)])
    colp = jnp.concatenate([edge_index[1], jnp.full((npad,), N, i32)])
    colA = colp.reshape(2 * NTILE, NB_W, BE)
    rowC = rowp.reshape(NTILE, NB_C, BE)
    colC = colp.reshape(NTILE, NB_C, BE)
    rowE = rowp.reshape(2 * NTILE, NB_W, BE)
    colE = colp.reshape(2 * NTILE, NB_W, BE)

    W2p = jnp.pad(W2, ((0, 0), (0, CW2 - OUT_DIM))).reshape(NCH, CW, CW2)
    b1r = b1.reshape(NCH, CW)
    b2p = jnp.pad(b2, (0, CW2 - OUT_DIM)).reshape(1, CW2)

    deg2 = _deg_call(colA).reshape(2, NH)
    g1 = _mm1_call(x, W1, deg2)
    s1 = _prop1_call(g1.reshape(NCH * NH, CW), rowC, colC)
    g2 = _mm2_call(s1.reshape(NCH, NH, CW), g1, deg2, b1r, W2p)
    s2 = _prop2_call(g2, rowE, colE)
    out16 = _fin_call(s2.reshape(2, NH, CW2), g2, deg2, b2p)
    return out16[:, :OUT_DIM]


# speedup vs baseline: 5.5046x; 5.5046x over previous
import jax
import jax.numpy as jnp
from jax import lax
from jax.experimental import pallas as pl
from jax.experimental.pallas import tpu as pltpu
from jax.experimental.pallas import tpu_sc as plsc

N = 10000
E = 160000
IN_DIM = 1433
HID = 512
OUT_DIM = 7

NTILE = 16
NH = 10240
ROWS_PT = NH // NTILE
EPAD = 163840
BE = 128
NB_C = EPAD // NTILE // BE
NB_W = EPAD // (2 * NTILE) // BE
CW = 128
NCH = HID // CW
CW2 = 128

_mesh = plsc.VectorSubcoreMesh(core_axis_name="c", subcore_axis_name="s")

f32 = jnp.float32
i32 = jnp.int32


def _zfill(ref, n):
    def body(j, _):
        ref[pl.ds(j * 16, 16)] = jnp.zeros((16,), f32)
        return 0
    lax.fori_loop(0, n // 16, body, 0)


def _deg_body(col_hbm, deg_hbm, colv, ones, zbuf, hist):
    c = lax.axis_index("c")
    s = lax.axis_index("s")
    w = c * NTILE + s
    def fo(j, _):
        ones[pl.ds(j * 16, 16)] = jnp.full((16,), 1.0, f32)
        return 0
    lax.fori_loop(0, BE // 16, fo, 0)
    _zfill(zbuf, ROWS_PT)
    pltpu.sync_copy(zbuf, hist.at[pl.ds(s * ROWS_PT, ROWS_PT)])
    pltpu.sync_copy(col_hbm.at[w], colv)
    plsc.subcore_barrier()
    def body(b, _):
        pltpu.sync_copy(ones, hist.at[colv.at[b]], add=True)
        return 0
    lax.fori_loop(0, NB_W, body, 0)
    plsc.subcore_barrier()
    pltpu.sync_copy(hist.at[pl.ds(s * ROWS_PT, ROWS_PT)],
                    deg_hbm.at[pl.ds(c * NH + s * ROWS_PT, ROWS_PT)])


def _deg_call(colA):
    return pl.kernel(
        _deg_body,
        out_type=jax.ShapeDtypeStruct((2 * NH,), f32),
        mesh=_mesh,
        scratch_types=[
            pltpu.VMEM((NB_W, BE), i32),
            pltpu.VMEM((BE,), f32),
            pltpu.VMEM((ROWS_PT,), f32),
            pltpu.VMEM_SHARED((NH,), f32),
        ],
    )(colA)


def _mm1_body(x_ref, w_ref, deg_ref, o_ref):
    d = deg_ref[:, 0] + deg_ref[:, 1] + 1.0
    dinv = lax.rsqrt(d)
    acc = jnp.dot(x_ref[...], w_ref[...], preferred_element_type=f32)
    o_ref[0] = acc * dinv[:, None]


def _mm1_call(x, W1, deg2):
    bn = 2000
    return pl.pallas_call(
        _mm1_body,
        grid=(N // bn, NCH),
        in_specs=[
            pl.BlockSpec((bn, IN_DIM), lambda i, k: (i, 0)),
            pl.BlockSpec((IN_DIM, CW), lambda i, k: (0, k)),
            pl.BlockSpec((bn, 2), lambda i, k: (i, 0)),
        ],
        out_specs=pl.BlockSpec((1, bn, CW), lambda i, k: (k, i, 0)),
        out_shape=jax.ShapeDtypeStruct((NCH, NH, CW), f32),
    )(x, W1, deg2)


def _prop1_body(g1_hbm, row_hbm, col_hbm, s1_hbm,
                rowv, colv, idxg, gbuf, zbuf, acc, sem):
    c = lax.axis_index("c")
    s = lax.axis_index("s")
    pltpu.sync_copy(row_hbm.at[s], rowv)
    pltpu.sync_copy(col_hbm.at[s], colv)
    def fz(r, _):
        def fz2(j, _):
            zbuf[r, pl.ds(j * 16, 16)] = jnp.zeros((16,), f32)
            return 0
        lax.fori_loop(0, CW // 16, fz2, 0)
        return 0
    lax.fori_loop(0, 64, fz, 0)
    for kl in range(2):
        k = c * 2 + kl
        off = k * NH
        def zc(q, _):
            pltpu.sync_copy(zbuf, acc.at[pl.ds(s * ROWS_PT + q * 64, 64)])
            return 0
        lax.fori_loop(0, ROWS_PT // 64, zc, 0)
        plsc.subcore_barrier()
        def body(b, _):
            for j in range(BE // 16):
                idxg[pl.ds(j * 16, 16)] = rowv[b, pl.ds(j * 16, 16)] + off
            pltpu.async_copy(g1_hbm.at[idxg], gbuf, sem).wait()
            pltpu.sync_copy(gbuf, acc.at[colv.at[b]], add=True)
            return 0
        lax.fori_loop(0, NB_C, body, 0)
        plsc.subcore_barrier()
        pltpu.sync_copy(acc.at[pl.ds(s * ROWS_PT, ROWS_PT)],
                        s1_hbm.at[pl.ds(off + s * ROWS_PT, ROWS_PT)])


def _prop1_call(g1f, rowC, colC):
    return pl.kernel(
        _prop1_body,
        out_type=jax.ShapeDtypeStruct((NCH * NH, CW), f32),
        mesh=_mesh,
        scratch_types=[
            pltpu.VMEM((NB_C, BE), i32),
            pltpu.VMEM((NB_C, BE), i32),
            pltpu.VMEM((BE,), i32),
            pltpu.VMEM((BE, CW), f32),
            pltpu.VMEM((64, CW), f32),
            pltpu.VMEM_SHARED((NH, CW), f32),
            pltpu.SemaphoreType.DMA,
        ],
    )(g1f, rowC, colC)


def _mm2_body(s1_ref, g1_ref, deg_ref, b1_ref, w2_ref, o_ref):
    d = deg_ref[:, 0] + deg_ref[:, 1] + 1.0
    dinv = lax.rsqrt(d)
    bn = d.shape[0]
    acc = jnp.zeros((bn, CW2), f32)
    for k in range(NCH):
        t = dinv[:, None] * (s1_ref[k] + g1_ref[k]) + b1_ref[k][None, :]
        t = jnp.maximum(t, 0.0)
        acc = acc + jnp.dot(t, w2_ref[k], preferred_element_type=f32)
    o_ref[...] = acc * dinv[:, None]


def _mm2_call(s1, g1, deg2, b1r, W2p):
    bn = 400
    return pl.pallas_call(
        _mm2_body,
        grid=(N // bn,),
        in_specs=[
            pl.BlockSpec((NCH, bn, CW), lambda i: (0, i, 0)),
            pl.BlockSpec((NCH, bn, CW), lambda i: (0, i, 0)),
            pl.BlockSpec((bn, 2), lambda i: (i, 0)),
            pl.BlockSpec((NCH, CW), lambda i: (0, 0)),
            pl.BlockSpec((NCH, CW, CW2), lambda i: (0, 0, 0)),
        ],
        out_specs=pl.BlockSpec((bn, CW2), lambda i: (i, 0)),
        out_shape=jax.ShapeDtypeStruct((N, CW2), f32),
    )(s1, g1, deg2, b1r, W2p)


def _prop2_body(g2_hbm, row_hbm, col_hbm, s2_hbm,
                rowv, colv, gbuf, zbuf, acc, sem):
    c = lax.axis_index("c")
    s = lax.axis_index("s")
    w = c * NTILE + s
    pltpu.sync_copy(row_hbm.at[w], rowv)
    pltpu.sync_copy(col_hbm.at[w], colv)
    def fz(r, _):
        zbuf[r, pl.ds(0, 16)] = jnp.zeros((16,), f32)
        return 0
    lax.fori_loop(0, 64, fz, 0)
    def zc(q, _):
        pltpu.sync_copy(zbuf, acc.at[pl.ds(s * ROWS_PT + q * 64, 64)])
        return 0
    lax.fori_loop(0, ROWS_PT // 64, zc, 0)
    plsc.subcore_barrier()
    def body(b, _):
        pltpu.async_copy(g2_hbm.at[rowv.at[b]], gbuf, sem).wait()
        pltpu.sync_copy(gbuf, acc.at[colv.at[b]], add=True)
        return 0
    lax.fori_loop(0, NB_W, body, 0)
    plsc.subcore_barrier()
    pltpu.sync_copy(acc.at[pl.ds(s * ROWS_PT, ROWS_PT)],
                    s2_hbm.at[pl.ds(c * NH + s * ROWS_PT, ROWS_PT)])


def _prop2_call(g2, rowE, colE):
    return pl.kernel(
        _prop2_body,
        out_type=jax.ShapeDtypeStruct((2 * NH, CW2), f32),
        mesh=_mesh,
        scratch_types=[
            pltpu.VMEM((NB_W, BE), i32),
            pltpu.VMEM((NB_W, BE), i32),
            pltpu.VMEM((BE, CW2), f32),
            pltpu.VMEM((64, CW2), f32),
            pltpu.VMEM_SHARED((NH, CW2), f32),
            pltpu.SemaphoreType.DMA,
        ],
    )(g2, rowE, colE)


def _fin_body(s2_ref, g2_ref, deg_ref, b2_ref, o_ref):
    d = deg_ref[:, 0] + deg_ref[:, 1] + 1.0
    dinv = lax.rsqrt(d)
    o_ref[...] = dinv[:, None] * (s2_ref[0] + s2_ref[1] + g2_ref[...]) \
        + b2_ref[0][None, :]


def _fin_call(s2, g2, deg2, b2p):
    bn = 400
    return pl.pallas_call(
        _fin_body,
        grid=(N // bn,),
        in_specs=[
            pl.BlockSpec((2, bn, CW2), lambda i: (0, i, 0)),
            pl.BlockSpec((bn, CW2), lambda i: (i, 0)),
            pl.BlockSpec((bn, 2), lambda i: (i, 0)),
            pl.BlockSpec((1, CW2), lambda i: (0, 0)),
        ],
        out_specs=pl.BlockSpec((bn, CW2), lambda i: (i, 0)),
        out_shape=jax.ShapeDtypeStruct((N, CW2), f32),
    )(s2, g2, deg2, b2p)


def kernel(x, edge_index, W1, b1, W2, b2):
    npad = EPAD - E
    rowp = jnp.concatenate([edge_index[0], jnp.zeros((npad,), i32)])
    colp = jnp.concatenate([edge_index[1], jnp.full((npad,), N, i32)])
    colA = colp.reshape(2 * NTILE, NB_W, BE)
    rowC = rowp.reshape(NTILE, NB_C, BE)
    colC = colp.reshape(NTILE, NB_C, BE)
    rowE = rowp.reshape(2 * NTILE, NB_W, BE)
    colE = colp.reshape(2 * NTILE, NB_W, BE)

    W2p = jnp.pad(W2, ((0, 0), (0, CW2 - OUT_DIM))).reshape(NCH, CW, CW2)
    b1r = b1.reshape(NCH, CW)
    b2p = jnp.pad(b2, (0, CW2 - OUT_DIM)).reshape(1, CW2)

    deg2 = _deg_call(colA).reshape(2, NH).T
    g1 = _mm1_call(x, W1, deg2)
    s1 = _prop1_call(g1.reshape(NCH * NH, CW), rowC, colC)
    g2 = _mm2_call(s1.reshape(NCH, NH, CW), g1, deg2, b1r, W2p)
    s2 = _prop2_call(g2, rowE, colE)
    out16 = _fin_call(s2.reshape(2, NH, CW2), g2, deg2, b2p)
    return out16[:, :OUT_DIM]
